# baseline (speedup 1.0000x reference)
import jax
import jax.numpy as jnp
from jax.experimental import pallas as pl
from jax.experimental.pallas import tpu as pltpu

_LN_EPS = 1e-5


def _fused_mlp_kernel(n_in):
    def body(*refs):
        xs = refs[:n_in]
        w1s = refs[n_in:2 * n_in]
        b1, g, be, w2, b2 = refs[2 * n_in:2 * n_in + 5]
        out = refs[-1]

        h = b1[...].astype(jnp.float32)
        acc = None
        for x, w in zip(xs, w1s):
            d = jnp.dot(x[...].astype(jnp.bfloat16), w[...],
                        preferred_element_type=jnp.float32)
            acc = d if acc is None else acc + d
        h = acc + h
        mu = jnp.mean(h, axis=-1, keepdims=True)
        hc = h - mu
        var = jnp.mean(hc * hc, axis=-1, keepdims=True)
        h = hc * jax.lax.rsqrt(var + _LN_EPS) * g[...] + be[...]
        h = jnp.maximum(h, 0.0)
        out[...] = jnp.dot(h.astype(jnp.bfloat16), w2[...],
                           preferred_element_type=jnp.float32) + b2[...]

    return body


def _fused_mlp(xs, p, n_splits, block_rows):
    n = xs[0].shape[0]
    din_each = [x.shape[1] for x in xs]
    w1 = p["W1"]
    w1s = []
    off = 0
    for d in din_each:
        w1s.append(w1[off:off + d].astype(jnp.bfloat16))
        off += d
    w2 = p["W2"].astype(jnp.bfloat16)
    b1 = p["b1"].reshape(1, -1)
    g = p["g"].reshape(1, -1)
    be = p["be"].reshape(1, -1)
    b2 = p["b2"].reshape(1, -1)
    dout = w2.shape[1]
    dh = w2.shape[0]

    grid = (n // block_rows,)
    x_specs = [pl.BlockSpec((block_rows, d), lambda i: (i, 0)) for d in din_each]
    w_specs = [pl.BlockSpec((d, dh), lambda i: (0, 0)) for d in din_each]
    vec_spec = pl.BlockSpec((1, dh), lambda i: (0, 0))
    w2_spec = pl.BlockSpec((dh, dout), lambda i: (0, 0))
    b2_spec = pl.BlockSpec((1, dout), lambda i: (0, 0))
    out_spec = pl.BlockSpec((block_rows, dout), lambda i: (i, 0))

    return pl.pallas_call(
        _fused_mlp_kernel(len(xs)),
        grid=grid,
        in_specs=x_specs + w_specs + [vec_spec, vec_spec, vec_spec, w2_spec, b2_spec],
        out_specs=out_spec,
        out_shape=jax.ShapeDtypeStruct((n, dout), jnp.float32),
    )(*xs, *w1s, b1, g, be, w2, b2)


def kernel(nodes, edges, params, eidx):
    bus = nodes["bus"]

    s, r = eidx["ac"][0], eidx["ac"][1]
    ue_ac = _fused_mlp([edges["ac"], bus[s], bus[r]], params["e_ac"], 3, 2000)
    r_ac = r

    s, r = eidx["tr"][0], eidx["tr"][1]
    ue_tr = _fused_mlp([edges["tr"], bus[s], bus[r]], params["e_tr"], 3, 2000)
    r_tr = r

    s, r = eidx["gen"][0], eidx["gen"][1]
    ue_gen = _fused_mlp([bus[s], nodes["generator"][r]], params["e_gen"], 2, 1000)
    r_gen = r

    s, r = eidx["load"][0], eidx["load"][1]
    ue_load = _fused_mlp([bus[s], nodes["load"][r]], params["e_load"], 2, 2000)
    r_load = r

    s, r = eidx["shunt"][0], eidx["shunt"][1]
    ue_shunt = _fused_mlp([bus[s], nodes["shunt"][r]], params["e_shunt"], 2, 2000)
    r_shunt = r

    ed = ue_ac.shape[1]
    msg_bus = jnp.zeros((bus.shape[0], ed), jnp.float32)
    msg_bus = msg_bus.at[r_ac].add(ue_ac)
    msg_bus = msg_bus.at[r_tr].add(ue_tr)
    msg_gen = jnp.zeros((nodes["generator"].shape[0], ed), jnp.float32).at[r_gen].add(ue_gen)
    msg_load = jnp.zeros((nodes["load"].shape[0], ed), jnp.float32).at[r_load].add(ue_load)
    msg_shunt = jnp.zeros((nodes["shunt"].shape[0], ed), jnp.float32).at[r_shunt].add(ue_shunt)

    nb = _fused_mlp([bus, msg_bus], params["n_bus"], 2, 2000)
    ng = _fused_mlp([nodes["generator"], msg_gen], params["n_generator"], 2, 1000)
    nl = _fused_mlp([nodes["load"], msg_load], params["n_load"], 2, 2000)
    ns = _fused_mlp([nodes["shunt"], msg_shunt], params["n_shunt"], 2, 2000)

    return (nb, ng, nl, ns, ue_ac, ue_tr, ue_gen, ue_load, ue_shunt)
